# baseline (speedup 1.0000x reference)
import functools

import jax
import jax.numpy as jnp
from jax import lax
from jax.experimental import pallas as pl
from jax.experimental.pallas import tpu as pltpu
from jax.experimental.pallas import tpu_sc as plsc

TOP_K = 8
EPS = 0.2
NUM_TOKENS = 32768
NUM_EXPERTS = 64

NUM_CORES = 2
NUM_SUBCORES = 16
LANES = 16
NUM_WORKERS = NUM_CORES * NUM_SUBCORES
ROWS_PER_W = NUM_TOKENS // NUM_WORKERS
GROUPS = ROWS_PER_W // LANES
CHUNK_WORDS = ROWS_PER_W * NUM_EXPERTS
OUT_WORDS = ROWS_PER_W * TOP_K
NSPLIT = 4
BLOCK = NUM_EXPERTS // NSPLIT

_T_POS = 1.0 - 2.0 * EPS
_T_NEG = 1.0 / (1.0 - 2.0 * EPS)


def _sc_body(logits_hbm, idx_hbm, val_hbm, chunk, lt0, et0, lt1, et1,
             idxs, vals):
    wid = lax.axis_index("s") * NUM_CORES + lax.axis_index("c")
    pltpu.sync_copy(logits_hbm.at[pl.ds(wid * CHUNK_WORDS, CHUNK_WORDS)], chunk)

    lanes = lax.iota(jnp.int32, 16)
    neg_inf = jnp.full((16,), -jnp.inf, jnp.float32)
    zero = jnp.zeros((16,), jnp.float32)
    izero = jnp.zeros((16,), jnp.int32)

    def merge_blocks(ms, ids):
        m, idxv = ms[0], ids[0]
        for b in range(1, NSPLIT):
            upd = ms[b] > m
            m = jnp.where(upd, ms[b], m)
            idxv = jnp.where(upd, ids[b], idxv)
        return m, idxv

    def one_group(g, lt, et):
        rows = g * LANES + lanes
        row_base = rows * NUM_EXPERTS
        ms = [neg_inf for _ in range(NSPLIT)]
        ids = [izero for _ in range(NSPLIT)]
        for b in range(NSPLIT):
            for j in range(b * BLOCK, (b + 1) * BLOCK):
                v = plsc.load_gather(chunk, [row_base + j])
                lt[pl.ds(j * LANES, LANES)] = v
                upd = v > ms[b]
                ms[b] = jnp.where(upd, v, ms[b])
                ids[b] = jnp.where(upd, jnp.int32(j), ids[b])
        m0, idxv = merge_blocks(ms, ids)
        m = m0
        for j in range(NUM_EXPERTS):
            s = pl.ds(j * LANES, LANES)
            et[s] = jnp.exp(lt[s] - m0)
        out_base = rows * TOP_K
        for i in range(TOP_K):
            t = jnp.where(m >= 0, _T_POS * m, _T_NEG * m)
            num = jnp.exp(m - m0)
            plsc.store_scatter(idxs, [out_base + i], idxv)
            plsc.store_scatter(lt, [idxv * LANES + lanes], neg_inf)
            accs = [zero for _ in range(NSPLIT)]
            nms = [neg_inf for _ in range(NSPLIT)]
            nids = [izero for _ in range(NSPLIT)]
            for b in range(NSPLIT):
                for j in range(b * BLOCK, (b + 1) * BLOCK):
                    s = pl.ds(j * LANES, LANES)
                    lv = lt[s]
                    ev = et[s]
                    accs[b] = accs[b] + jnp.where(lv >= t, ev, zero)
                    upd = lv > nms[b]
                    nms[b] = jnp.where(upd, lv, nms[b])
                    nids[b] = jnp.where(upd, jnp.int32(j), nids[b])
            acc = (accs[0] + accs[1]) + (accs[2] + accs[3])
            val = num / (acc + num)
            plsc.store_scatter(vals, [out_base + i], val)
            m, idxv = merge_blocks(nms, nids)

    def pair_body(p, carry):
        one_group(2 * p, lt0, et0)
        one_group(2 * p + 1, lt1, et1)
        return carry

    lax.fori_loop(0, GROUPS // 2, pair_body, jnp.int32(0))

    pltpu.sync_copy(idxs, idx_hbm.at[pl.ds(wid * OUT_WORDS, OUT_WORDS)])
    pltpu.sync_copy(vals, val_hbm.at[pl.ds(wid * OUT_WORDS, OUT_WORDS)])


_sc_call = functools.partial(
    pl.kernel,
    out_type=(
        jax.ShapeDtypeStruct((NUM_TOKENS * TOP_K,), jnp.int32),
        jax.ShapeDtypeStruct((NUM_TOKENS * TOP_K,), jnp.float32),
    ),
    mesh=plsc.VectorSubcoreMesh(core_axis_name="c", subcore_axis_name="s"),
    compiler_params=pltpu.CompilerParams(needs_layout_passes=False),
    scratch_types=[
        pltpu.VMEM((CHUNK_WORDS,), jnp.float32),
        pltpu.VMEM((NUM_EXPERTS * LANES,), jnp.float32),
        pltpu.VMEM((NUM_EXPERTS * LANES,), jnp.float32),
        pltpu.VMEM((NUM_EXPERTS * LANES,), jnp.float32),
        pltpu.VMEM((NUM_EXPERTS * LANES,), jnp.float32),
        pltpu.VMEM((OUT_WORDS,), jnp.int32),
        pltpu.VMEM((OUT_WORDS,), jnp.float32),
    ],
)(_sc_body)


@jax.jit
def kernel(router_logits):
    flat = jnp.reshape(router_logits.astype(jnp.float32), (-1,))
    idx_flat, val_flat = _sc_call(flat)
    return (jnp.reshape(idx_flat, (NUM_TOKENS, TOP_K)),
            jnp.reshape(val_flat, (NUM_TOKENS, TOP_K)))


# speedup vs baseline: 1.6153x; 1.6153x over previous
import functools

import jax
import jax.numpy as jnp
from jax import lax
from jax.experimental import pallas as pl
from jax.experimental.pallas import tpu as pltpu
from jax.experimental.pallas import tpu_sc as plsc

TOP_K = 8
EPS = 0.2
NUM_TOKENS = 32768
NUM_EXPERTS = 64

NUM_CORES = 2
NUM_SUBCORES = 16
LANES = 16
NUM_WORKERS = NUM_CORES * NUM_SUBCORES
ROWS_PER_W = NUM_TOKENS // NUM_WORKERS
GROUPS = ROWS_PER_W // LANES
CHUNK_WORDS = ROWS_PER_W * NUM_EXPERTS
OUT_WORDS = ROWS_PER_W * TOP_K

_T_POS = 1.0 - 2.0 * EPS
_T_NEG = 1.0 / (1.0 - 2.0 * EPS)
_IDX_MASK = 63


def _sc_body(logits_hbm, idx_hbm, val_hbm, chunk, lt, et, idxs, vals):
    wid = lax.axis_index("s") * NUM_CORES + lax.axis_index("c")
    pltpu.sync_copy(logits_hbm.at[pl.ds(wid * CHUNK_WORDS, CHUNK_WORDS)], chunk)

    lanes = lax.iota(jnp.int32, 16)
    neg_inf = jnp.full((16,), -jnp.inf, jnp.float32)
    zero = jnp.zeros((16,), jnp.float32)
    izero = jnp.zeros((16,), jnp.int32)

    def group_body(g, carry):
        rows = g * LANES + lanes
        row_base = rows * NUM_EXPERTS
        m = neg_inf
        idxv = izero
        for j in range(NUM_EXPERTS):
            v = plsc.load_gather(chunk, [row_base + j])
            lt[pl.ds(j * LANES, LANES)] = v
            upd = v > m
            m = jnp.where(upd, v, m)
            idxv = jnp.where(upd, jnp.int32(j), idxv)
        m0 = m
        ml = m0
        for j in range(NUM_EXPERTS):
            s = pl.ds(j * LANES, LANES)
            e = jnp.exp(lt[s] - m0)
            eb = lax.bitcast_convert_type(e, jnp.int32)
            eb = (eb & jnp.int32(~_IDX_MASK)) | jnp.int32(_IDX_MASK - j)
            et[s] = lax.bitcast_convert_type(eb, jnp.float32)
        out_base = rows * TOP_K
        num = jnp.full((16,), 1.0, jnp.float32)
        for i in range(TOP_K):
            t = jnp.where(ml >= 0, _T_POS * ml, _T_NEG * ml)
            tau = jnp.exp(t - m0)
            taub = lax.bitcast_convert_type(tau, jnp.int32) & jnp.int32(~_IDX_MASK)
            tau = lax.bitcast_convert_type(taub, jnp.float32)
            plsc.store_scatter(idxs, [out_base + i], idxv)
            plsc.store_scatter(et, [idxv * LANES + lanes], zero)
            accs = [zero] * 4
            nms = [zero] * 4
            for b in range(4):
                for j in range(b * 16, (b + 1) * 16):
                    ev = et[pl.ds(j * LANES, LANES)]
                    accs[b] = accs[b] + jnp.where(ev >= tau, ev, zero)
                    nms[b] = jnp.maximum(nms[b], ev)
            acc = (accs[0] + accs[1]) + (accs[2] + accs[3])
            nm = jnp.maximum(jnp.maximum(nms[0], nms[1]),
                             jnp.maximum(nms[2], nms[3]))
            val = num / (acc + num)
            plsc.store_scatter(vals, [out_base + i], val)
            nb = lax.bitcast_convert_type(nm, jnp.int32)
            idxv = jnp.int32(_IDX_MASK) - (nb & jnp.int32(_IDX_MASK))
            num = lax.bitcast_convert_type(nb & jnp.int32(~_IDX_MASK),
                                           jnp.float32)
            if i + 1 < TOP_K:
                ml = plsc.load_gather(lt, [idxv * LANES + lanes])
        return carry

    lax.fori_loop(0, GROUPS, group_body, jnp.int32(0))

    pltpu.sync_copy(idxs, idx_hbm.at[pl.ds(wid * OUT_WORDS, OUT_WORDS)])
    pltpu.sync_copy(vals, val_hbm.at[pl.ds(wid * OUT_WORDS, OUT_WORDS)])


_sc_call = functools.partial(
    pl.kernel,
    out_type=(
        jax.ShapeDtypeStruct((NUM_TOKENS * TOP_K,), jnp.int32),
        jax.ShapeDtypeStruct((NUM_TOKENS * TOP_K,), jnp.float32),
    ),
    mesh=plsc.VectorSubcoreMesh(core_axis_name="c", subcore_axis_name="s"),
    compiler_params=pltpu.CompilerParams(needs_layout_passes=False),
    scratch_types=[
        pltpu.VMEM((CHUNK_WORDS,), jnp.float32),
        pltpu.VMEM((NUM_EXPERTS * LANES,), jnp.float32),
        pltpu.VMEM((NUM_EXPERTS * LANES,), jnp.float32),
        pltpu.VMEM((OUT_WORDS,), jnp.int32),
        pltpu.VMEM((OUT_WORDS,), jnp.float32),
    ],
)(_sc_body)


@jax.jit
def kernel(router_logits):
    flat = jnp.reshape(router_logits.astype(jnp.float32), (-1,))
    idx_flat, val_flat = _sc_call(flat)
    return (jnp.reshape(idx_flat, (NUM_TOKENS, TOP_K)),
            jnp.reshape(val_flat, (NUM_TOKENS, TOP_K)))
